# baseline (speedup 1.0000x reference)
import functools

import jax
import jax.numpy as jnp
from jax import lax
from jax.experimental import pallas as pl
from jax.experimental.pallas import tpu as pltpu, tpu_sc as plsc

N = 1024
P = 4096
K2 = 8
BQ = 128
G = N // BQ

NC = 2
NS = 16
NW = NC * NS
RW = N // NW
CH = 8
NCH = RW // CH
LANES = 16


def _quaternion_apply(q, v):
    w = q[..., :1]
    xyz = q[..., 1:]
    t = 2.0 * jnp.cross(xyz, v)
    return v + w * t + jnp.cross(xyz, t)


def _argmin_body(eq_ref, s_ref, out_ref):
    m = None
    for k in range(K2):
        dx = eq_ref[:, 3 * k + 0][:, None] - s_ref[0:1, :]
        dy = eq_ref[:, 3 * k + 1][:, None] - s_ref[1:2, :]
        dz = eq_ref[:, 3 * k + 2][:, None] - s_ref[2:3, :]
        d2 = dx * dx + dy * dy + dz * dz
        m = d2 if m is None else jnp.minimum(m, d2)
    minval = jnp.min(m, axis=1, keepdims=True)
    iota = lax.broadcasted_iota(jnp.int32, m.shape, 1)
    idx = jnp.min(jnp.where(m == minval, iota, P), axis=1)
    out_ref[0, 0, :] = idx


def _ids_tc(eq2, s2T):
    out = pl.pallas_call(
        _argmin_body,
        grid=(G,),
        in_specs=[
            pl.BlockSpec((BQ, 24), lambda i: (i, 0)),
            pl.BlockSpec((3, P), lambda i: (0, 0)),
        ],
        out_specs=pl.BlockSpec((1, 1, BQ), lambda i: (i, 0, 0)),
        out_shape=jax.ShapeDtypeStruct((G, 1, BQ), jnp.int32),
    )(eq2, s2T)
    return out.reshape(N)


def _sc_gather_body(ids_hbm, covmat_hbm, out_hbm,
                    ids_v, rows_a, rows_b, out_a, out_b, gsem, osem):
    wid = lax.axis_index("s") * NC + lax.axis_index("c")
    base = wid * RW
    pltpu.sync_copy(ids_hbm, ids_v)
    bufs = (rows_a, rows_b)
    obufs = (out_a, out_b)

    def gather_chunk(c, buf):
        return pltpu.async_copy(
            covmat_hbm.at[ids_v.at[pl.ds(base + c * CH, CH)]], buf, gsem)

    gather_chunk(0, bufs[0])
    for c in range(NCH):
        buf = bufs[c % 2]
        ob = obufs[c % 2]
        pltpu.make_async_copy(
            covmat_hbm.at[ids_v.at[pl.ds(base + c * CH, CH)]], buf, gsem
        ).wait()
        if c + 1 < NCH:
            gather_chunk(c + 1, bufs[(c + 1) % 2])
        if c >= 2:
            pltpu.make_async_copy(
                ob, out_hbm.at[pl.ds(base + (c - 2) * CH, CH)], osem).wait()
        for r in range(CH):
            row_idx = jnp.full((LANES,), r, jnp.int32)

            def jbody(j, carry, buf=buf, ob=ob, r=r, row_idx=row_idx):
                off = pl.multiple_of(j * LANES, LANES)
                cols = ids_v[pl.ds(off, LANES)]
                vals = plsc.load_gather(buf, [row_idx, cols])
                ob[r, pl.ds(off, LANES)] = vals
                return carry

            lax.fori_loop(0, N // LANES, jbody, 0)
        pltpu.async_copy(ob, out_hbm.at[pl.ds(base + c * CH, CH)], osem)
    for c in (NCH - 2, NCH - 1):
        pltpu.make_async_copy(
            obufs[c % 2], out_hbm.at[pl.ds(base + c * CH, CH)], osem).wait()


def _gather_sc(ids, covmat):
    mesh = plsc.VectorSubcoreMesh(core_axis_name="c", subcore_axis_name="s")
    f = pl.kernel(
        _sc_gather_body,
        mesh=mesh,
        compiler_params=pltpu.CompilerParams(needs_layout_passes=False),
        out_type=jax.ShapeDtypeStruct((N, N), jnp.float32),
        scratch_types=[
            pltpu.VMEM((N,), jnp.int32),
            pltpu.VMEM((CH, P), jnp.float32),
            pltpu.VMEM((CH, P), jnp.float32),
            pltpu.VMEM((CH, N), jnp.float32),
            pltpu.VMEM((CH, N), jnp.float32),
            pltpu.SemaphoreType.DMA,
            pltpu.SemaphoreType.DMA,
        ],
    )
    return f(ids, covmat)


def kernel(s2_query_points, s2_fz_pts, laue_q, covmat):
    eqiv = _quaternion_apply(laue_q[None, :, :], s2_query_points[:, None, :])
    eqiv = jnp.concatenate([eqiv, -eqiv], axis=1)
    eq2 = eqiv.reshape(N, 3 * K2)
    s2T = s2_fz_pts.T
    ids = _ids_tc(eq2, s2T)
    return _gather_sc(ids, covmat)


# speedup vs baseline: 1.1370x; 1.1370x over previous
import functools

import jax
import jax.numpy as jnp
from jax import lax
from jax.experimental import pallas as pl
from jax.experimental.pallas import tpu as pltpu, tpu_sc as plsc

N = 1024
P = 4096
K2 = 8
BQ = 128
G = N // BQ

NC = 2
NS = 16
NW = NC * NS
RW = N // NW
CH = 8
NCH = RW // CH
LANES = 16


def _quaternion_apply(q, v):
    w = q[..., :1]
    xyz = q[..., 1:]
    t = 2.0 * jnp.cross(xyz, v)
    return v + w * t + jnp.cross(xyz, t)


def _argmin_body(eq_ref, s_ref, out_ref):
    m = None
    for k in range(K2):
        dx = eq_ref[:, 3 * k + 0][:, None] - s_ref[0:1, :]
        dy = eq_ref[:, 3 * k + 1][:, None] - s_ref[1:2, :]
        dz = eq_ref[:, 3 * k + 2][:, None] - s_ref[2:3, :]
        d2 = dx * dx + dy * dy + dz * dz
        m = d2 if m is None else jnp.minimum(m, d2)
    minval = jnp.min(m, axis=1, keepdims=True)
    iota = lax.broadcasted_iota(jnp.int32, m.shape, 1)
    idx = jnp.min(jnp.where(m == minval, iota, P), axis=1)
    out_ref[0, 0, :] = idx


def _ids_tc(eq2, s2T):
    out = pl.pallas_call(
        _argmin_body,
        grid=(G,),
        in_specs=[
            pl.BlockSpec((BQ, 24), lambda i: (i, 0)),
            pl.BlockSpec((3, P), lambda i: (0, 0)),
        ],
        out_specs=pl.BlockSpec((1, 1, BQ), lambda i: (i, 0, 0)),
        out_shape=jax.ShapeDtypeStruct((G, 1, BQ), jnp.int32),
    )(eq2, s2T)
    return out.reshape(N)


def _sc_gather_body(ids_hbm, covmat_hbm, out_hbm,
                    ids_v, rows_a, rows_b, out_a, out_b, gsem, osem):
    wid = lax.axis_index("s") * NC + lax.axis_index("c")
    base = wid * RW
    pltpu.sync_copy(ids_hbm, ids_v)
    bufs = (rows_a, rows_b)
    obufs = (out_a, out_b)

    def gather_chunk(c, buf):
        return pltpu.async_copy(
            covmat_hbm.at[ids_v.at[pl.ds(base + c * CH, CH)]], buf, gsem)

    gather_chunk(0, bufs[0])
    for c in range(NCH):
        buf = bufs[c % 2]
        ob = obufs[c % 2]
        pltpu.make_async_copy(
            covmat_hbm.at[ids_v.at[pl.ds(base + c * CH, CH)]], buf, gsem
        ).wait()
        if c + 1 < NCH:
            gather_chunk(c + 1, bufs[(c + 1) % 2])
        if c >= 2:
            pltpu.make_async_copy(
                ob, out_hbm.at[pl.ds(base + (c - 2) * CH, CH)], osem).wait()
        def jbody(j, carry, buf=buf, ob=ob):
            off = pl.multiple_of(j * LANES, LANES)
            cols = ids_v[pl.ds(off, LANES)]
            for r in range(CH):
                vals = plsc.load_gather(buf, [jnp.full((LANES,), r, jnp.int32), cols])
                ob[r, pl.ds(off, LANES)] = vals
            return carry

        lax.fori_loop(0, N // LANES, jbody, 0)
        pltpu.async_copy(ob, out_hbm.at[pl.ds(base + c * CH, CH)], osem)
    for c in (NCH - 2, NCH - 1):
        pltpu.make_async_copy(
            obufs[c % 2], out_hbm.at[pl.ds(base + c * CH, CH)], osem).wait()


def _gather_sc(ids, covmat):
    mesh = plsc.VectorSubcoreMesh(core_axis_name="c", subcore_axis_name="s")
    f = pl.kernel(
        _sc_gather_body,
        mesh=mesh,
        compiler_params=pltpu.CompilerParams(needs_layout_passes=False),
        out_type=jax.ShapeDtypeStruct((N, N), jnp.float32),
        scratch_types=[
            pltpu.VMEM((N,), jnp.int32),
            pltpu.VMEM((CH, P), jnp.float32),
            pltpu.VMEM((CH, P), jnp.float32),
            pltpu.VMEM((CH, N), jnp.float32),
            pltpu.VMEM((CH, N), jnp.float32),
            pltpu.SemaphoreType.DMA,
            pltpu.SemaphoreType.DMA,
        ],
    )
    return f(ids, covmat)


def kernel(s2_query_points, s2_fz_pts, laue_q, covmat):
    eqiv = _quaternion_apply(laue_q[None, :, :], s2_query_points[:, None, :])
    eqiv = jnp.concatenate([eqiv, -eqiv], axis=1)
    eq2 = eqiv.reshape(N, 3 * K2)
    s2T = s2_fz_pts.T
    ids = _ids_tc(eq2, s2T)
    return _gather_sc(ids, covmat)


# speedup vs baseline: 1.4010x; 1.2322x over previous
import functools

import jax
import jax.numpy as jnp
from jax import lax
from jax.experimental import pallas as pl
from jax.experimental.pallas import tpu as pltpu, tpu_sc as plsc

N = 1024
P = 4096
K2 = 8
BQ = 128
G = N // BQ

NC = 2
NS = 16
NW = NC * NS
RW = N // NW
CH = 8
NCH = RW // CH
LANES = 16


def _quaternion_apply(q, v):
    w = q[..., :1]
    xyz = q[..., 1:]
    t = 2.0 * jnp.cross(xyz, v)
    return v + w * t + jnp.cross(xyz, t)


CW = 512


def _argmin_body(eq_ref, s_ref, out_ref):
    bestv = None
    besti = None
    for pc in range(0, P, CW):
        s0 = s_ref[0:1, pc:pc + CW]
        s1 = s_ref[1:2, pc:pc + CW]
        s2 = s_ref[2:3, pc:pc + CW]
        m = None
        for k in range(K2):
            dx = eq_ref[:, 3 * k + 0][:, None] - s0
            dy = eq_ref[:, 3 * k + 1][:, None] - s1
            dz = eq_ref[:, 3 * k + 2][:, None] - s2
            d2 = dx * dx + dy * dy + dz * dz
            m = d2 if m is None else jnp.minimum(m, d2)
        cmin = jnp.min(m, axis=1, keepdims=True)
        iota = lax.broadcasted_iota(jnp.int32, m.shape, 1) + pc
        cidx = jnp.min(jnp.where(m == cmin, iota, P), axis=1, keepdims=True)
        if bestv is None:
            bestv, besti = cmin, cidx
        else:
            upd = cmin < bestv
            besti = jnp.where(upd, cidx, besti)
            bestv = jnp.where(upd, cmin, bestv)
    out_ref[0, 0, :] = besti[:, 0]


def _ids_tc(eq2, s2T):
    out = pl.pallas_call(
        _argmin_body,
        grid=(G,),
        in_specs=[
            pl.BlockSpec((BQ, 24), lambda i: (i, 0)),
            pl.BlockSpec((3, P), lambda i: (0, 0)),
        ],
        out_specs=pl.BlockSpec((1, 1, BQ), lambda i: (i, 0, 0)),
        out_shape=jax.ShapeDtypeStruct((G, 1, BQ), jnp.int32),
    )(eq2, s2T)
    return out.reshape(N)


def _sc_gather_body(ids_hbm, covmat_hbm, out_hbm,
                    ids_v, rows_a, rows_b, out_a, out_b, gsem, osem):
    wid = lax.axis_index("s") * NC + lax.axis_index("c")
    base = wid * RW
    pltpu.sync_copy(ids_hbm, ids_v)
    bufs = (rows_a, rows_b)
    obufs = (out_a, out_b)

    def gather_chunk(c, buf):
        return pltpu.async_copy(
            covmat_hbm.at[ids_v.at[pl.ds(base + c * CH, CH)]], buf, gsem)

    gather_chunk(0, bufs[0])
    for c in range(NCH):
        buf = bufs[c % 2]
        ob = obufs[c % 2]
        pltpu.make_async_copy(
            covmat_hbm.at[ids_v.at[pl.ds(base + c * CH, CH)]], buf, gsem
        ).wait()
        if c + 1 < NCH:
            gather_chunk(c + 1, bufs[(c + 1) % 2])
        if c >= 2:
            pltpu.make_async_copy(
                ob, out_hbm.at[pl.ds(base + (c - 2) * CH, CH)], osem).wait()
        def jbody(j, carry, buf=buf, ob=ob):
            off = pl.multiple_of(j * LANES, LANES)
            cols = ids_v[pl.ds(off, LANES)]
            for r in range(CH):
                vals = plsc.load_gather(buf, [jnp.full((LANES,), r, jnp.int32), cols])
                ob[r, pl.ds(off, LANES)] = vals
            return carry

        lax.fori_loop(0, N // LANES, jbody, 0)
        pltpu.async_copy(ob, out_hbm.at[pl.ds(base + c * CH, CH)], osem)
    for c in (NCH - 2, NCH - 1):
        pltpu.make_async_copy(
            obufs[c % 2], out_hbm.at[pl.ds(base + c * CH, CH)], osem).wait()


def _gather_sc(ids, covmat):
    mesh = plsc.VectorSubcoreMesh(core_axis_name="c", subcore_axis_name="s")
    f = pl.kernel(
        _sc_gather_body,
        mesh=mesh,
        compiler_params=pltpu.CompilerParams(needs_layout_passes=False),
        out_type=jax.ShapeDtypeStruct((N, N), jnp.float32),
        scratch_types=[
            pltpu.VMEM((N,), jnp.int32),
            pltpu.VMEM((CH, P), jnp.float32),
            pltpu.VMEM((CH, P), jnp.float32),
            pltpu.VMEM((CH, N), jnp.float32),
            pltpu.VMEM((CH, N), jnp.float32),
            pltpu.SemaphoreType.DMA,
            pltpu.SemaphoreType.DMA,
        ],
    )
    return f(ids, covmat)


def kernel(s2_query_points, s2_fz_pts, laue_q, covmat):
    eqiv = _quaternion_apply(laue_q[None, :, :], s2_query_points[:, None, :])
    eqiv = jnp.concatenate([eqiv, -eqiv], axis=1)
    eq2 = eqiv.reshape(N, 3 * K2)
    s2T = s2_fz_pts.T
    ids = _ids_tc(eq2, s2T)
    return _gather_sc(ids, covmat)
